# Initial kernel scaffold; baseline (speedup 1.0000x reference)
import jax
import jax.numpy as jnp
from jax.experimental import pallas as pl


def kernel(x, type_id, edge_index, edge_attr, k_w, k_b, q_w, q_b, v_w, v_b, a_w, a_b, relation_pri, relation_att, relation_msg, skip):
    raise NotImplementedError("write your pallas kernel here")


# speedup vs baseline: 16.3135x; 16.3135x over previous
import functools
import math

import jax
import jax.numpy as jnp
from jax import lax
from jax.experimental import pallas as pl
from jax.experimental.pallas import tpu as pltpu
from jax.experimental.pallas import tpu_sc as plsc

N = 10000
E = 320000
C = 128
H = 8
DK = 16
T = 4
R = 4
NR = N * R
NP = 10048
NW = 32
CB = 256
SUB = CB // 128
EPW = 10240
EP = EPW * NW
NCH = EPW // CB
ROWS_PW = EPW // 128
SQRT_DK = math.sqrt(DK)

_f32 = jnp.float32
_i32 = jnp.int32


def _proj_body(x_ref, w_ref, b_ref, kt_ref, q_ref, vt_ref):
    y = jnp.dot(x_ref[...], w_ref[...], preferred_element_type=_f32) + b_ref[...]
    kt_ref[...] = y[:, :512]
    q_ref[...] = y[:, 512:1024]
    vt_ref[...] = y[:, 1024:]


def _run_proj(x, wbig, bbig):
    nb = 10
    blk = N // nb
    return pl.pallas_call(
        _proj_body,
        grid=(nb,),
        in_specs=[
            pl.BlockSpec((blk, C), lambda i: (i, 0)),
            pl.BlockSpec((C, 3 * 512), lambda i: (0, 0)),
            pl.BlockSpec((1, 3 * 512), lambda i: (0, 0)),
        ],
        out_specs=[
            pl.BlockSpec((blk, 512), lambda i: (i, 0)),
            pl.BlockSpec((blk, 512), lambda i: (i, 0)),
            pl.BlockSpec((blk, 512), lambda i: (i, 0)),
        ],
        out_shape=[jax.ShapeDtypeStruct((N, 512), _f32)] * 3,
    )(x, wbig, bbig)


def _edge_body(gk_ref, gq_ref, gv_ref, s_ref, x_ref, ex_ref, w_ref):
    z = jnp.dot(gk_ref[...] * gq_ref[...], s_ref[...],
                preferred_element_type=_f32)
    ex = jnp.exp(z)
    ex_ref[...] = ex
    w_ref[...] = gv_ref[...] * jnp.dot(ex[:, :16], x_ref[...],
                                       preferred_element_type=_f32)


def _run_edge(gk, gq, gv, s16, x16):
    blk = 2048
    nb = EP // blk
    return pl.pallas_call(
        _edge_body,
        grid=(nb,),
        in_specs=[
            pl.BlockSpec((blk, C), lambda i: (i, 0)),
            pl.BlockSpec((blk, C), lambda i: (i, 0)),
            pl.BlockSpec((blk, C), lambda i: (i, 0)),
            pl.BlockSpec((C, C), lambda i: (0, 0)),
            pl.BlockSpec((16, C), lambda i: (0, 0)),
        ],
        out_specs=[
            pl.BlockSpec((blk, C), lambda i: (i, 0)),
            pl.BlockSpec((blk, C), lambda i: (i, 0)),
        ],
        out_shape=[jax.ShapeDtypeStruct((EP, C), _f32),
                   jax.ShapeDtypeStruct((EP, C), _f32)],
    )(gk, gq, gv, s16, x16)


def _final_body(s0_ref, s1_ref, d0_ref, d1_ref, tid_ref, x_ref, x16_ref,
                awt_ref, ab_ref, skip_ref, out_ref):
    den = d0_ref[...] + d1_ref[...]
    invd = 1.0 / (den[:, :16] + 1e-16)
    icnt = 1.0 / jnp.maximum(den[:, 8:9], 1.0)
    factor = jnp.dot(invd, x16_ref[...],
                     preferred_element_type=_f32) * icnt
    x0 = (s0_ref[...] + s1_ref[...]) * factor
    tid = tid_ref[...]
    acc = jnp.zeros_like(x0)
    aval = jnp.zeros_like(icnt)
    for t in range(T):
        m = tid == t
        proj = jnp.dot(x0, awt_ref[:, t * C:(t + 1) * C],
                       preferred_element_type=_f32) + ab_ref[t:t + 1, :]
        acc = jnp.where(m, proj, acc)
        aval = jnp.where(m, skip_ref[0, t], aval)
    alpha = 1.0 / (1.0 + jnp.exp(-aval))
    out_ref[...] = acc * alpha + x_ref[...] * (1.0 - alpha)


def _run_final(sum0, sum1, den0, den1, tid2d, x, x16, awt, ab, skip2d):
    nb = 10
    blk = N // nb
    return pl.pallas_call(
        _final_body,
        grid=(nb,),
        in_specs=[
            pl.BlockSpec((blk, C), lambda i: (i, 0)),
            pl.BlockSpec((blk, C), lambda i: (i, 0)),
            pl.BlockSpec((blk, C), lambda i: (i, 0)),
            pl.BlockSpec((blk, C), lambda i: (i, 0)),
            pl.BlockSpec((blk, 1), lambda i: (i, 0)),
            pl.BlockSpec((blk, C), lambda i: (i, 0)),
            pl.BlockSpec((16, C), lambda i: (0, 0)),
            pl.BlockSpec((C, T * C), lambda i: (0, 0)),
            pl.BlockSpec((T, C), lambda i: (0, 0)),
            pl.BlockSpec((1, T), lambda i: (0, 0)),
        ],
        out_specs=pl.BlockSpec((blk, C), lambda i: (i, 0)),
        out_shape=jax.ShapeDtypeStruct((N, C), _f32),
    )(sum0, sum1, den0, den1, tid2d, x, x16, awt, ab, skip2d)


_MESH = plsc.VectorSubcoreMesh(core_axis_name="c", subcore_axis_name="s")


@functools.partial(
    pl.kernel,
    mesh=_MESH,
    out_type=(jax.ShapeDtypeStruct((EP, C), _f32),) * 3,
    scratch_types=[
        pltpu.VMEM((SUB, 128), _i32),
        pltpu.VMEM((SUB, 128), _i32),
        pltpu.VMEM((CB, C), _f32),
        pltpu.VMEM((CB, C), _f32),
        pltpu.VMEM((CB, C), _f32),
        pltpu.SemaphoreType.DMA,
    ],
)
def _sc_gather(kt_hbm, qt_hbm, vt_hbm, ik_hbm, iq_hbm,
               gk_hbm, gq_hbm, gv_hbm, ikv, iqv, bk, bq, bv, sem):
    wid = lax.axis_index("s") * 2 + lax.axis_index("c")
    row0 = wid * ROWS_PW

    def chunk(ci, carry):
        r0 = row0 + ci * SUB
        e0 = r0 * 128
        pltpu.sync_copy(ik_hbm.at[pl.ds(r0, SUB)], ikv)
        pltpu.sync_copy(iq_hbm.at[pl.ds(r0, SUB)], iqv)
        for j in range(SUB):
            pltpu.async_copy(kt_hbm.at[ikv.at[j]],
                             bk.at[pl.ds(j * 128, 128)], sem).wait()
            pltpu.async_copy(qt_hbm.at[iqv.at[j]],
                             bq.at[pl.ds(j * 128, 128)], sem).wait()
            pltpu.async_copy(vt_hbm.at[ikv.at[j]],
                             bv.at[pl.ds(j * 128, 128)], sem).wait()
        pltpu.sync_copy(bk, gk_hbm.at[pl.ds(e0, CB)])
        pltpu.sync_copy(bq, gq_hbm.at[pl.ds(e0, CB)])
        pltpu.sync_copy(bv, gv_hbm.at[pl.ds(e0, CB)])
        return carry

    lax.fori_loop(0, NCH, chunk, 0)


def _make_sc_scatter(width):
    @functools.partial(
        pl.kernel,
        mesh=_MESH,
        out_type=jax.ShapeDtypeStruct((2, NP, width), _f32),
        scratch_types=[
            pltpu.VMEM((SUB, 128), _i32),
            pltpu.VMEM((CB, width), _f32),
            pltpu.VMEM_SHARED((NP, width), _f32),
        ],
    )
    def _sc_scatter(val_hbm, dst_hbm, zero_hbm, acc_hbm, idxv, valv, sh):
        cidx = lax.axis_index("c")
        sidx = lax.axis_index("s")

        @pl.when(sidx == 0)
        def _init():
            pltpu.sync_copy(zero_hbm, sh)

        plsc.subcore_barrier()
        wid = sidx * 2 + cidx
        row0 = wid * ROWS_PW

        def chunk(ci, carry):
            r0 = row0 + ci * SUB
            e0 = r0 * 128
            pltpu.sync_copy(dst_hbm.at[pl.ds(r0, SUB)], idxv)
            pltpu.sync_copy(val_hbm.at[pl.ds(e0, CB)], valv)
            for j in range(SUB):
                pltpu.sync_copy(valv.at[pl.ds(j * 128, 128)],
                                sh.at[idxv.at[j]], add=True)
            return carry

        lax.fori_loop(0, NCH, chunk, 0)
        plsc.subcore_barrier()

        @pl.when(sidx == 0)
        def _writeout():
            pltpu.sync_copy(sh, acc_hbm.at[cidx])

    return _sc_scatter


_sc_scatter128 = _make_sc_scatter(C)


def kernel(x, type_id, edge_index, edge_attr, k_w, k_b, q_w, q_b, v_w, v_b,
           a_w, a_b, relation_pri, relation_att, relation_msg, skip):
    x = x.astype(_f32)

    scale = (relation_pri / SQRT_DK).astype(_f32)
    A = relation_att.astype(_f32) * scale[:, :, None, None]
    M = relation_msg.astype(_f32)
    kw4 = k_w.astype(_f32).reshape(R, H, DK, C)
    vw4 = v_w.astype(_f32).reshape(R, H, DK, C)
    Wkt = jnp.einsum('rhdi,rhdj->rihj', kw4, A).reshape(R, C, C)
    bkt = jnp.einsum('rhd,rhdj->rhj', k_b.astype(_f32).reshape(R, H, DK),
                     A).reshape(R, C)
    Wvt = jnp.einsum('rhdi,rhdj->rihj', vw4, M).reshape(R, C, C)
    bvt = jnp.einsum('rhd,rhdj->rhj', v_b.astype(_f32).reshape(R, H, DK),
                     M).reshape(R, C)
    Wq = jnp.transpose(q_w.astype(_f32), (0, 2, 1))
    bq = q_b.astype(_f32)

    wbig = jnp.concatenate([
        jnp.transpose(Wkt, (1, 0, 2)).reshape(C, R * C),
        jnp.transpose(Wq, (1, 0, 2)).reshape(C, R * C),
        jnp.transpose(Wvt, (1, 0, 2)).reshape(C, R * C),
    ], axis=1)
    bbig = jnp.concatenate([bkt.reshape(-1), bq.reshape(-1),
                            bvt.reshape(-1)]).reshape(1, 3 * 512)

    src = edge_index[0].astype(_i32)
    dst = edge_index[1].astype(_i32)
    attr = edge_attr.astype(_i32)
    pad = EP - E
    ik = jnp.pad(src * R + attr, (0, pad)).reshape(EP // 128, 128)
    iq = jnp.pad(dst * R + attr, (0, pad)).reshape(EP // 128, 128)
    dstp = jnp.pad(dst, (0, pad), constant_values=N).reshape(EP // 128, 128)

    s16 = (jnp.arange(C)[:, None] // DK == jnp.arange(C)[None, :]
           ).astype(_f32)
    x16 = (jnp.arange(16)[:, None] == jnp.arange(C)[None, :] // DK
           ).astype(_f32)
    zero128 = jnp.zeros((NP, C), _f32)

    kt_n, q_n, vt_n = _run_proj(x, wbig, bbig)
    kt_t = kt_n.reshape(NR, C)
    q_t = q_n.reshape(NR, C)
    vt_t = vt_n.reshape(NR, C)

    gk, gq, gv = _sc_gather(kt_t, q_t, vt_t, ik, iq)

    ex16, weighted = _run_edge(gk, gq, gv, s16, x16)

    den = _sc_scatter128(ex16, dstp, zero128)
    summ = _sc_scatter128(weighted, dstp, zero128)
    den0, den1 = den[0], den[1]
    sum0, sum1 = summ[0], summ[1]

    awt = jnp.transpose(a_w.astype(_f32), (2, 0, 1)).reshape(C, T * C)
    ab = a_b.astype(_f32)
    skip2d = skip.astype(_f32).reshape(1, T)
    tid2d = type_id.astype(_i32).reshape(N, 1)
    out = _run_final(sum0, sum1, den0, den1, tid2d, x, x16, awt, ab, skip2d)
    return out
